# baseline (speedup 1.0000x reference)
import functools

import jax
import jax.numpy as jnp
from jax import lax
from jax.experimental import pallas as pl
from jax.experimental.pallas import tpu as pltpu
from jax.experimental.pallas import tpu_sc as plsc

_NORM = 11.0
_INV_NORM = 1.0 / 11.0
_LN2 = 0.6931471805599453
_C3 = 1.0 / 3.0
_C5 = 0.2
_C7 = 1.0 / 7.0

_W = 512
_STRIP = 8 * _W
_NW = 32
_TOTAL = 2 * 1 * 16 * 512 * 512
_N_STRIPS = _TOTAL // _STRIP
_STRIPS_PER_W = _N_STRIPS // _NW
_NGROUPS = 4


def _log1p_over_norm(q):
    y = q + 1.0
    b = lax.bitcast_convert_type(y, jnp.int32)
    e = (b >> 23) - 127
    m = lax.bitcast_convert_type((b & 0x7FFFFF) | 0x3F800000, jnp.float32)
    ef = e.astype(jnp.float32)
    z = (m - 1.0) / (m + 1.0)
    z2 = z * z
    p = (_C7 * z2 + _C5) * z2 + _C3
    p = p * z2 + 1.0
    return (ef * _LN2 + 2.0 * z * p) * _INV_NORM


def _expm1_norm(v):
    return jnp.maximum(jnp.exp(v * _NORM) - 1.0, 0.0)


def _bitonic_ce(v, i, j, up):
    a, b = v[i], v[j]
    lo = jnp.minimum(a, b)
    hi = jnp.maximum(a, b)
    if up:
        v[i], v[j] = lo, hi
    else:
        v[i], v[j] = hi, lo


def _bitonic_sort64(v):
    n = 64
    k = 2
    while k <= n:
        j = k // 2
        while j >= 1:
            for i in range(n):
                ixj = i ^ j
                if ixj > i:
                    _bitonic_ce(v, i, ixj, (i & k) == 0)
            j //= 2
        k *= 2


@functools.partial(
    pl.kernel,
    out_type=jax.ShapeDtypeStruct((_TOTAL,), jnp.float32),
    mesh=plsc.VectorSubcoreMesh(core_axis_name="c", subcore_axis_name="s"),
    scratch_types=[
        pltpu.VMEM((_STRIP,), jnp.float32),
        pltpu.VMEM((_STRIP,), jnp.float32),
        pltpu.VMEM((_STRIP,), jnp.float32),
        pltpu.VMEM((64 * 16,), jnp.float32),
        pltpu.VMEM((64 * 16,), jnp.float32),
    ],
    compiler_params=pltpu.CompilerParams(needs_layout_passes=False),
)
def _wf_kernel(pred_hbm, inp_hbm, out_hbm, pred_v, inp_v, out_v, p_buf, s_buf):
    nc = 2
    wid = lax.axis_index("s") * nc + lax.axis_index("c")
    io8 = lax.iota(jnp.int32, 16) * 8

    def strip_body(i, carry):
        off = (wid * _STRIPS_PER_W + i) * _STRIP
        pltpu.sync_copy(pred_hbm.at[pl.ds(off, _STRIP)], pred_v)
        pltpu.sync_copy(inp_hbm.at[pl.ds(off, _STRIP)], inp_v)

        def grp(g, c2):
            gbase = g * 128
            idx = [io8 + (gbase + (e >> 3) * _W + (e & 7)) for e in range(64)]
            acc = []
            for e in range(64):
                iv = _expm1_norm(plsc.load_gather(inp_v, [idx[e]]))
                acc.append(iv)
            width = 64
            while width > 1:
                width //= 2
                for e in range(width):
                    acc[e] = acc[e] + acc[e + width]
            S = acc[0]
            v = []
            for e in range(64):
                pv = _expm1_norm(plsc.load_gather(pred_v, [idx[e]]))
                p_buf[pl.ds(e * 16, 16)] = pv
                v.append(pv)
            _bitonic_sort64(v)
            for e in range(64):
                s_buf[pl.ds(e * 16, 16)] = v[e]
            d = 1
            while d < 64:
                for e in range(64 - d):
                    v_new = v[e] + v[e + d]
                    v[e] = v_new
                d *= 2
            msum = jnp.zeros((16,), jnp.float32)
            rho = jnp.zeros((16,), jnp.float32)
            for e in range(64):
                suf = v[e]
                s_e = s_buf[pl.ds(e * 16, 16)]
                kf = float(64 - e)
                active = (kf * s_e) > (suf - S)
                msum = msum + jnp.where(active, s_e, 0.0)
                rho = rho + jnp.where(active, 1.0, 0.0)
            maxp = v[63]
            theta = jnp.where(rho > 0.5, (msum - S) / jnp.maximum(rho, 1.0),
                              maxp - S)
            for e in range(64):
                pv = p_buf[pl.ds(e * 16, 16)]
                q = jnp.maximum(pv - theta, 0.0)
                plsc.store_scatter(out_v, [idx[e]], _log1p_over_norm(q))
            return c2

        lax.fori_loop(0, _NGROUPS, grp, 0, unroll=False)
        pltpu.sync_copy(out_v, out_hbm.at[pl.ds(off, _STRIP)])
        return carry

    lax.fori_loop(0, _STRIPS_PER_W, strip_body, 0, unroll=False)


def kernel(pred_log_norm, input_mosaic_log_norm):
    shape = pred_log_norm.shape
    pf = pred_log_norm.reshape(-1)
    nf = input_mosaic_log_norm.reshape(-1)
    return _wf_kernel(pf, nf).reshape(shape)


# speedup vs baseline: 3.5565x; 3.5565x over previous
import functools

import jax
import jax.numpy as jnp
from jax import lax
from jax.experimental import pallas as pl
from jax.experimental.pallas import tpu as pltpu
from jax.experimental.pallas import tpu_sc as plsc

_NORM = 11.0
_INV_NORM = 1.0 / 11.0
_LN2 = 0.6931471805599453
_SQRT2 = 1.4142135623730951
_C3 = 1.0 / 3.0
_C5 = 0.2
_C7 = 1.0 / 7.0

_W = 512
_STRIP = 8 * _W
_NW = 32
_TOTAL = 2 * 1 * 16 * 512 * 512
_N_STRIPS = _TOTAL // _STRIP
_STRIPS_PER_W = _N_STRIPS // _NW
_BLOCKS_PER_STRIP = _W // 8


def _merge16(a, b):
    rb = jnp.flip(b)
    lo = jnp.minimum(a, rb)
    hi = jnp.maximum(a, rb)
    return jnp.sort(lo), jnp.sort(hi)


def _sort64(v0, v1, v2, v3):
    s0, s1, s2, s3 = jnp.sort(v0), jnp.sort(v1), jnp.sort(v2), jnp.sort(v3)
    a0, a1 = _merge16(s0, s1)
    b0, b1 = _merge16(s2, s3)
    rb0, rb1 = jnp.flip(b1), jnp.flip(b0)
    l0 = jnp.minimum(a0, rb0)
    l1 = jnp.minimum(a1, rb1)
    h0 = jnp.maximum(a0, rb0)
    h1 = jnp.maximum(a1, rb1)
    m0 = jnp.minimum(l0, l1)
    m1 = jnp.maximum(l0, l1)
    m2 = jnp.minimum(h0, h1)
    m3 = jnp.maximum(h0, h1)
    return jnp.sort(m0), jnp.sort(m1), jnp.sort(m2), jnp.sort(m3)


def _log1p_over_norm(q):
    y = q + 1.0
    b = lax.bitcast_convert_type(y, jnp.int32)
    e = (b >> 23) - 127
    m = lax.bitcast_convert_type((b & 0x7FFFFF) | 0x3F800000, jnp.float32)
    big = m > _SQRT2
    m = jnp.where(big, m * 0.5, m)
    ef = e.astype(jnp.float32) + jnp.where(big, 1.0, 0.0)
    z = (m - 1.0) / (m + 1.0)
    z2 = z * z
    p = (_C7 * z2 + _C5) * z2 + _C3
    p = p * z2 + 1.0
    return (ef * _LN2 + 2.0 * z * p) * _INV_NORM


def _expm1_norm(v):
    return jnp.maximum(jnp.exp(v * _NORM) - 1.0, 0.0)


@functools.partial(
    pl.kernel,
    out_type=jax.ShapeDtypeStruct((_TOTAL,), jnp.float32),
    mesh=plsc.VectorSubcoreMesh(core_axis_name="c", subcore_axis_name="s"),
    scratch_types=[
        pltpu.VMEM((_STRIP,), jnp.float32),
        pltpu.VMEM((_STRIP,), jnp.float32),
        pltpu.VMEM((_STRIP,), jnp.float32),
    ],
    compiler_params=pltpu.CompilerParams(needs_layout_passes=False),
)
def _wf_kernel(pred_hbm, inp_hbm, out_hbm, pred_v, inp_v, out_v):
    nc = 2
    wid = lax.axis_index("s") * nc + lax.axis_index("c")
    io = lax.iota(jnp.int32, 16)
    row = io >> 3
    col = io & 7
    base = [(2 * k + row) * _W + col for k in range(4)]
    kf = (io + 1).astype(jnp.float32)
    inv_k = [1.0 / (kf + (16.0 * j)) for j in range(4)]

    def strip_body(i, carry):
        off = (wid * _STRIPS_PER_W + i) * _STRIP
        pltpu.sync_copy(pred_hbm.at[pl.ds(off, _STRIP)], pred_v)
        pltpu.sync_copy(inp_hbm.at[pl.ds(off, _STRIP)], inp_v)

        @plsc.parallel_loop(0, _BLOCKS_PER_STRIP, 1, unroll=2)
        def blk(bi):
            idx = [base[k] + bi * 8 for k in range(4)]
            pv = [_expm1_norm(plsc.load_gather(pred_v, [idx[k]])) for k in range(4)]
            iv = [_expm1_norm(plsc.load_gather(inp_v, [idx[k]])) for k in range(4)]
            S = jnp.sum((iv[0] + iv[1]) + (iv[2] + iv[3]))
            m0, m1, m2, m3 = _sort64(pv[0], pv[1], pv[2], pv[3])
            u = [jnp.flip(m3), jnp.flip(m2), jnp.flip(m1), jnp.flip(m0)]
            acc_m = jnp.zeros((16,), jnp.float32)
            acc_c = jnp.zeros((16,), jnp.float32)
            carry_s = 0.0
            for j in range(4):
                cj = plsc.cumsum(u[j]) + carry_s
                carry_s = carry_s + jnp.sum(u[j])
                t = (cj - S) * inv_k[j]
                mask = u[j] > t
                acc_m = acc_m + jnp.where(mask, u[j], 0.0)
                acc_c = acc_c + jnp.where(mask, 1.0, 0.0)
            msum = jnp.broadcast_to(jnp.sum(acc_m), (16,))
            rho = jnp.broadcast_to(jnp.sum(acc_c), (16,))
            maxp = jnp.broadcast_to(jnp.max(u[0]), (16,))
            theta = jnp.where(rho > 0.5, (msum - S) / jnp.maximum(rho, 1.0),
                              maxp - S)
            for k in range(4):
                q = jnp.maximum(pv[k] - theta, 0.0)
                plsc.store_scatter(out_v, [idx[k]], _log1p_over_norm(q))

        pltpu.sync_copy(out_v, out_hbm.at[pl.ds(off, _STRIP)])
        return carry

    lax.fori_loop(0, _STRIPS_PER_W, strip_body, 0, unroll=False)


def kernel(pred_log_norm, input_mosaic_log_norm):
    shape = pred_log_norm.shape
    pf = pred_log_norm.reshape(-1)
    nf = input_mosaic_log_norm.reshape(-1)
    return _wf_kernel(pf, nf).reshape(shape)
